# baseline (speedup 1.0000x reference)
import functools
import jax
import jax.numpy as jnp
from jax import lax
from jax.experimental import pallas as pl
from jax.experimental.pallas import tpu as pltpu
import jax.experimental.pallas.tpu_sc as plsc

VOCAB = 30522
HID = 768
B = 32
S = 512
LV = 100
EPS = 1e-5

NSPLIT = 8
QB = B // NSPLIT
GCHUNK = 32
SPLICE = 128
TCG = 4
NCH = 14
SC_ROWS = NCH * GCHUNK


_NSHARE = 32 // QB
_NITER = -(-NCH // _NSHARE)


def _sc_gather(ids_hbm, word_hbm, out_hbm, idx_a, idx_b, rows_a, rows_b,
               gsem_a, gsem_b, ssem_a, ssem_b):
  wid = lax.axis_index("s") * 2 + lax.axis_index("c")
  b_loc = wid // _NSHARE
  q = wid % _NSHARE

  def _ci(i):
    return jnp.minimum(q * _NITER + i, NCH - 1)

  idx_v = (idx_a, idx_b)
  rows_v = (rows_a, rows_b)
  gsem = (gsem_a, gsem_b)
  ssem = (ssem_a, ssem_b)
  gd = [None, None]
  st = [None, None]
  for i in range(_NITER):
    p = i % 2
    if st[p] is not None:
      st[p].wait()
    ci = _ci(i)
    c = ci + 2 * (ci >= 1).astype(jnp.int32)
    pltpu.sync_copy(ids_hbm.at[pl.ds(b_loc * S + c * GCHUNK, GCHUNK)],
                    idx_v[p])
    gd[p] = pltpu.async_copy(word_hbm.at[idx_v[p]], rows_v[p], gsem[p])
    pp = (i - 1) % 2
    if gd[pp] is not None and i >= 1:
      gd[pp].wait()
      st[pp] = pltpu.async_copy(
          rows_v[pp],
          out_hbm.at[pl.ds(b_loc * SC_ROWS + _ci(i - 1) * GCHUNK, GCHUNK)],
          ssem[pp])
  lastp = (_NITER - 1) % 2
  gd[lastp].wait()
  st[lastp] = pltpu.async_copy(
      rows_v[lastp],
      out_hbm.at[pl.ds(b_loc * SC_ROWS + _ci(_NITER - 1) * GCHUNK, GCHUNK)],
      ssem[lastp])
  for d in st:
    if d is not None:
      d.wait()


def _ln(emb, gam, bet):
  u = jnp.mean(emb, axis=1, keepdims=True)
  d = emb - u
  var = jnp.mean(d * d, axis=1, keepdims=True)
  return gam * (d * lax.rsqrt(var + EPS)) + bet


def _tc_body(voff, dummy_ref, a_ref, visf_ref, vispe_ref, pos_ref, typ_ref,
             gam_ref, bet_ref, o_ref):
  g = pl.program_id(0)
  typ = typ_ref[...]
  gam = gam_ref[...]
  bet = bet_ref[...]
  r = lax.broadcasted_iota(jnp.int32, (SPLICE, 1), 0)
  c = lax.broadcasted_iota(jnp.int32, (1, LV), 1)
  perm = (r == c + 1).astype(jnp.float32)
  m0 = jnp.logical_and(r[:32] >= 1, r[:32] <= LV)
  m2 = r[96:SPLICE] <= LV
  for j in range(TCG):
    a = a_ref[j]
    b = voff + g * TCG + j
    vs = visf_ref[:, b, :] + vispe_ref[:, b, :]
    vss = jax.lax.dot(perm, vs,
                      preferred_element_type=jnp.float32)
    seg0 = jnp.where(m0, vss[:32], a[0:32] + pos_ref[0:32]) + typ
    o_ref[j, 0:32, :] = _ln(seg0, gam, bet)
    seg1 = vss[32:96] + typ
    o_ref[j, 32:96, :] = _ln(seg1, gam, bet)
    seg2 = jnp.where(m2, vss[96:SPLICE], a[32:64] + pos_ref[96:SPLICE]) + typ
    o_ref[j, 96:SPLICE, :] = _ln(seg2, gam, bet)
    seg3 = a[64:SC_ROWS] + pos_ref[SPLICE:] + typ
    o_ref[j, SPLICE:, :] = _ln(seg3, gam, bet)


def _sc_call(ids_q, word_emb):
  mesh = plsc.VectorSubcoreMesh(core_axis_name="c", subcore_axis_name="s")
  return pl.kernel(
      _sc_gather,
      out_type=jax.ShapeDtypeStruct((QB * SC_ROWS, HID), jnp.float32),
      mesh=mesh,
      scratch_types=[
          pltpu.VMEM((GCHUNK,), jnp.int32),
          pltpu.VMEM((GCHUNK,), jnp.int32),
          pltpu.VMEM((GCHUNK, HID), jnp.float32),
          pltpu.VMEM((GCHUNK, HID), jnp.float32),
          pltpu.SemaphoreType.DMA,
          pltpu.SemaphoreType.DMA,
          pltpu.SemaphoreType.DMA,
          pltpu.SemaphoreType.DMA,
      ],
  )(ids_q, word_emb)


VBLK = 8


def _tc_call(dummy, a_q, qi, vt_f, vt_p, pos_emb, typ, gam, bet, alias):

  voff = (qi * QB) % VBLK
  vblk_idx = (qi * QB) // VBLK
  return pl.pallas_call(
      functools.partial(_tc_body, voff),
      grid=(QB // TCG,),
      in_specs=[
          pl.BlockSpec(memory_space=pl.ANY),
          pl.BlockSpec((TCG, SC_ROWS, HID), lambda b: (b, 0, 0)),
          pl.BlockSpec((LV, VBLK, HID), lambda b, q=vblk_idx: (0, q, 0)),
          pl.BlockSpec((LV, VBLK, HID), lambda b, q=vblk_idx: (0, q, 0)),
          pl.BlockSpec((S, HID), lambda b: (0, 0)),
          pl.BlockSpec((1, HID), lambda b: (0, 0)),
          pl.BlockSpec((1, HID), lambda b: (0, 0)),
          pl.BlockSpec((1, HID), lambda b: (0, 0)),
      ],
      out_specs=pl.BlockSpec((TCG, S, HID),
                             lambda b, q=qi: (b + q * (QB // TCG), 0, 0)),
      out_shape=jax.ShapeDtypeStruct((B, S, HID), jnp.float32),
      input_output_aliases={0: 0} if alias else {},
  )(dummy, a_q, vt_f, vt_p, pos_emb, typ, gam, bet)


@jax.jit
def kernel(vis_feats, vis_pe, input_ids, word_emb, pos_emb, type_emb,
           ln_gamma, ln_beta):
  ids = input_ids.reshape(-1).astype(jnp.int32)

  gs = [_sc_call(ids[qi * QB * S:(qi + 1) * QB * S], word_emb)
        for qi in range(NSPLIT)]

  typ = type_emb[0:1]
  gam = ln_gamma.reshape(1, HID)
  bet = ln_beta.reshape(1, HID)
  vt_f = vis_feats.transpose(1, 0, 2)
  vt_p = vis_pe.transpose(1, 0, 2)

  out = None
  for qi in range(NSPLIT):
    a_q = gs[qi].reshape(QB, SC_ROWS, HID)
    dummy = a_q if out is None else out
    out = _tc_call(dummy, a_q, qi, vt_f, vt_p, pos_emb, typ, gam, bet,
                   alias=out is not None)
  return out


# speedup vs baseline: 1.2471x; 1.2471x over previous
import functools
import jax
import jax.numpy as jnp
from jax import lax
from jax.experimental import pallas as pl
from jax.experimental.pallas import tpu as pltpu
import jax.experimental.pallas.tpu_sc as plsc

VOCAB = 30522
HID = 768
B = 32
S = 512
LV = 100
EPS = 1e-5

NSPLIT = 2
QB = B // NSPLIT
GCHUNK = 32
SPLICE = 128
TCG = 4
NCH = 14
SC_ROWS = NCH * GCHUNK


_NSHARE = 32 // QB
_NITER = -(-NCH // _NSHARE)


def _sc_gather(ids_hbm, word_hbm, out_hbm, idx_a, idx_b, rows_a, rows_b,
               gsem_a, gsem_b, ssem_a, ssem_b):
  wid = lax.axis_index("s") * 2 + lax.axis_index("c")
  b_loc = wid // _NSHARE
  q = wid % _NSHARE

  def _ci(i):
    return jnp.minimum(q * _NITER + i, NCH - 1)

  idx_v = (idx_a, idx_b)
  rows_v = (rows_a, rows_b)
  gsem = (gsem_a, gsem_b)
  ssem = (ssem_a, ssem_b)
  gd = [None, None]
  st = [None, None]
  for i in range(_NITER):
    p = i % 2
    if st[p] is not None:
      st[p].wait()
    ci = _ci(i)
    c = ci + 2 * (ci >= 1).astype(jnp.int32)
    pltpu.sync_copy(ids_hbm.at[pl.ds(b_loc * S + c * GCHUNK, GCHUNK)],
                    idx_v[p])
    gd[p] = pltpu.async_copy(word_hbm.at[idx_v[p]], rows_v[p], gsem[p])
    pp = (i - 1) % 2
    if gd[pp] is not None and i >= 1:
      gd[pp].wait()
      st[pp] = pltpu.async_copy(
          rows_v[pp],
          out_hbm.at[pl.ds(b_loc * SC_ROWS + _ci(i - 1) * GCHUNK, GCHUNK)],
          ssem[pp])
  lastp = (_NITER - 1) % 2
  gd[lastp].wait()
  st[lastp] = pltpu.async_copy(
      rows_v[lastp],
      out_hbm.at[pl.ds(b_loc * SC_ROWS + _ci(_NITER - 1) * GCHUNK, GCHUNK)],
      ssem[lastp])
  for d in st:
    if d is not None:
      d.wait()


def _ln(emb, gam, bet):
  u = jnp.mean(emb, axis=1, keepdims=True)
  d = emb - u
  var = jnp.mean(d * d, axis=1, keepdims=True)
  return gam * (d * lax.rsqrt(var + EPS)) + bet


def _tc_body(voff, dummy_ref, a_ref, visf_ref, vispe_ref, pos_ref, typ_ref,
             gam_ref, bet_ref, o_ref):
  g = pl.program_id(0)
  typ = typ_ref[...]
  gam = gam_ref[...]
  bet = bet_ref[...]
  r = lax.broadcasted_iota(jnp.int32, (SPLICE, 1), 0)
  c = lax.broadcasted_iota(jnp.int32, (1, LV), 1)
  perm = (r == c + 1).astype(jnp.float32)
  m0 = jnp.logical_and(r[:32] >= 1, r[:32] <= LV)
  m2 = r[96:SPLICE] <= LV
  for j in range(TCG):
    a = a_ref[j]
    b = voff + g * TCG + j
    vs = visf_ref[:, b, :] + vispe_ref[:, b, :]
    vss = jax.lax.dot(perm, vs,
                      preferred_element_type=jnp.float32)
    seg0 = jnp.where(m0, vss[:32], a[0:32] + pos_ref[0:32]) + typ
    o_ref[j, 0:32, :] = _ln(seg0, gam, bet)
    seg1 = vss[32:96] + typ
    o_ref[j, 32:96, :] = _ln(seg1, gam, bet)
    seg2 = jnp.where(m2, vss[96:SPLICE], a[32:64] + pos_ref[96:SPLICE]) + typ
    o_ref[j, 96:SPLICE, :] = _ln(seg2, gam, bet)
    seg3 = a[64:SC_ROWS] + pos_ref[SPLICE:] + typ
    o_ref[j, SPLICE:, :] = _ln(seg3, gam, bet)


def _sc_call(ids_q, word_emb):
  mesh = plsc.VectorSubcoreMesh(core_axis_name="c", subcore_axis_name="s")
  return pl.kernel(
      _sc_gather,
      out_type=jax.ShapeDtypeStruct((QB * SC_ROWS, HID), jnp.float32),
      mesh=mesh,
      scratch_types=[
          pltpu.VMEM((GCHUNK,), jnp.int32),
          pltpu.VMEM((GCHUNK,), jnp.int32),
          pltpu.VMEM((GCHUNK, HID), jnp.float32),
          pltpu.VMEM((GCHUNK, HID), jnp.float32),
          pltpu.SemaphoreType.DMA,
          pltpu.SemaphoreType.DMA,
          pltpu.SemaphoreType.DMA,
          pltpu.SemaphoreType.DMA,
      ],
  )(ids_q, word_emb)


VBLK = 16


def _tc_call(dummy, a_q, qi, vt_f, vt_p, pos_emb, typ, gam, bet, alias):

  voff = (qi * QB) % VBLK
  vblk_idx = (qi * QB) // VBLK
  return pl.pallas_call(
      functools.partial(_tc_body, voff),
      grid=(QB // TCG,),
      in_specs=[
          pl.BlockSpec(memory_space=pl.ANY),
          pl.BlockSpec((TCG, SC_ROWS, HID), lambda b: (b, 0, 0)),
          pl.BlockSpec((LV, VBLK, HID), lambda b, q=vblk_idx: (0, q, 0)),
          pl.BlockSpec((LV, VBLK, HID), lambda b, q=vblk_idx: (0, q, 0)),
          pl.BlockSpec((S, HID), lambda b: (0, 0)),
          pl.BlockSpec((1, HID), lambda b: (0, 0)),
          pl.BlockSpec((1, HID), lambda b: (0, 0)),
          pl.BlockSpec((1, HID), lambda b: (0, 0)),
      ],
      out_specs=pl.BlockSpec((TCG, S, HID),
                             lambda b, q=qi: (b + q * (QB // TCG), 0, 0)),
      out_shape=jax.ShapeDtypeStruct((B, S, HID), jnp.float32),
      input_output_aliases={0: 0} if alias else {},
  )(dummy, a_q, vt_f, vt_p, pos_emb, typ, gam, bet)


@jax.jit
def kernel(vis_feats, vis_pe, input_ids, word_emb, pos_emb, type_emb,
           ln_gamma, ln_beta):
  ids = input_ids.reshape(-1).astype(jnp.int32)

  gs = [_sc_call(ids[qi * QB * S:(qi + 1) * QB * S], word_emb)
        for qi in range(NSPLIT)]

  typ = type_emb[0:1]
  gam = ln_gamma.reshape(1, HID)
  bet = ln_beta.reshape(1, HID)
  vt_f = vis_feats.transpose(1, 0, 2)
  vt_p = vis_pe.transpose(1, 0, 2)

  out = None
  for qi in range(NSPLIT):
    a_q = gs[qi].reshape(QB, SC_ROWS, HID)
    dummy = a_q if out is None else out
    out = _tc_call(dummy, a_q, qi, vt_f, vt_p, pos_emb, typ, gam, bet,
                   alias=out is not None)
  return out
